# Initial kernel scaffold; baseline (speedup 1.0000x reference)
import functools

import jax
import jax.numpy as jnp
from jax import lax
from jax.experimental import pallas as pl
from jax.experimental.pallas import tpu as pltpu
from jax.experimental.pallas import tpu_sc as plsc

B, C, N = 4, 32, 1024
KT, K = 50, 20
L = KT * C
BM = 320
BN = 256


def _dot(a, b, ca, cb):
    return lax.dot_general(a, b, dimension_numbers=(((ca,), (cb,)), ((), ())),
                           preferred_element_type=jnp.float32)


def _ln(x, g, b):
    mu = jnp.mean(x, axis=-1, keepdims=True)
    var = jnp.mean((x - mu) ** 2, axis=-1, keepdims=True)
    return (x - mu) / jnp.sqrt(var + 1e-5) * g + b


def _knn_body(xf_ref, xb_ref, idx_ref):
    b = pl.program_id(0)
    xf = xf_ref[0]
    xb = xb_ref[0]
    xsq = jnp.sum(xf * xf, axis=0, keepdims=True)
    xx = _dot(xb, xf, 0, 0)
    sqb = jnp.sum(xb * xb, axis=0)[:, None]
    dist = 2.0 * xx - sqb - xsq
    lanes = lax.broadcasted_iota(jnp.int32, (BN, N), 1)
    cols = lax.broadcasted_iota(jnp.int32, (BN, KT), 1)
    boff = b * N

    def step(t, carry):
        d, res = carry
        m = jnp.max(d, axis=1, keepdims=True)
        cand = jnp.where(d == m, lanes, N)
        a = jnp.min(cand, axis=1, keepdims=True)
        res = jnp.where(cols == t, a + boff, res)
        d = jnp.where(lanes == a, -jnp.inf, d)
        return d, res

    _, res = lax.fori_loop(0, KT, step,
                           (dist, jnp.zeros((BN, KT), jnp.int32)))
    idx_ref[0] = res


def _knn(x):
    return pl.pallas_call(
        _knn_body,
        grid=(B, N // BN),
        in_specs=[pl.BlockSpec((1, C, N), lambda b, i: (b, 0, 0)),
                  pl.BlockSpec((1, C, BN), lambda b, i: (b, 0, i))],
        out_specs=pl.BlockSpec((1, BN, KT), lambda b, i: (b, i, 0)),
        out_shape=jax.ShapeDtypeStruct((B, N, KT), jnp.int32),
    )(x, x)


def _sc_gather(table, idx2d):
    info = plsc.get_sparse_core_info()
    nw = info.num_cores * info.num_subcores
    R = idx2d.shape[0]
    rpw = R // nw

    @functools.partial(
        pl.kernel,
        mesh=plsc.VectorSubcoreMesh(core_axis_name="c", subcore_axis_name="s"),
        out_type=jax.ShapeDtypeStruct((R * 128, C), jnp.float32),
        scratch_types=[pltpu.VMEM((rpw, 128), jnp.int32),
                       pltpu.VMEM((128, C), jnp.float32),
                       pltpu.SemaphoreType.DMA],
    )
    def k(table_hbm, idx_hbm, out_hbm, idx_v, rows_v, sem):
        wid = lax.axis_index("s") * info.num_cores + lax.axis_index("c")
        pltpu.sync_copy(idx_hbm.at[pl.ds(wid * rpw, rpw)], idx_v)

        def body(ci, _):
            pltpu.async_copy(table_hbm.at[idx_v.at[ci]], rows_v, sem).wait()
            pltpu.sync_copy(rows_v,
                            out_hbm.at[pl.ds((wid * rpw + ci) * 128, 128)])
            return 0

        lax.fori_loop(0, rpw, body, 0)

    return k(table, idx2d)


def _center_body(feat_ref, xt_ref, out_ref):
    out_ref[0] = feat_ref[0] - xt_ref[0][:, None, :]


def _center(feat4, xt3):
    return pl.pallas_call(
        _center_body,
        grid=(B, N // BN),
        in_specs=[pl.BlockSpec((1, BN, KT, C), lambda b, i: (b, i, 0, 0)),
                  pl.BlockSpec((1, BN, C), lambda b, i: (b, i, 0))],
        out_specs=pl.BlockSpec((1, BN, KT, C), lambda b, i: (b, i, 0, 0)),
        out_shape=jax.ShapeDtypeStruct((B, N, KT, C), jnp.float32),
    )(feat4, xt3)


def _kv_body(xc_ref, wk_ref, bk_ref, wv_ref, bv_ref, k_ref, v_ref):
    xc = xc_ref[0]
    k_ref[0] = _dot(xc, wk_ref[...], 1, 1) + bk_ref[...]
    v_ref[0] = _dot(xc, wv_ref[...], 1, 1) + bv_ref[...]


def _kv(flat, Wk, bk, Wv, bv):
    return pl.pallas_call(
        _kv_body,
        grid=(B, L // BM),
        in_specs=[pl.BlockSpec((1, BM, N), lambda b, i: (b, i, 0)),
                  pl.BlockSpec((N, N), lambda b, i: (0, 0)),
                  pl.BlockSpec((1, N), lambda b, i: (0, 0)),
                  pl.BlockSpec((N, N), lambda b, i: (0, 0)),
                  pl.BlockSpec((1, N), lambda b, i: (0, 0))],
        out_specs=[pl.BlockSpec((1, BM, N), lambda b, i: (b, i, 0)),
                   pl.BlockSpec((1, BM, N), lambda b, i: (b, i, 0))],
        out_shape=[jax.ShapeDtypeStruct((B, L, N), jnp.float32),
                   jax.ShapeDtypeStruct((B, L, N), jnp.float32)],
    )(flat, Wk, bk, Wv, bv)


def _attn_body(xc_ref, kk_ref, vv_ref, wq_ref, bq_ref, wd_ref, bd_ref,
               g1_ref, b1_ref, g2_ref, b2_ref, wfc_ref, bfc_ref, s_ref):
    xc = xc_ref[0]
    q = _dot(xc, wq_ref[...], 1, 1) + bq_ref[...]
    logits = _dot(q, kk_ref[0], 1, 1) * (1.0 / 32.0)
    m = jnp.max(logits, axis=1, keepdims=True)
    e = jnp.exp(logits - m)
    attn = e / jnp.sum(e, axis=1, keepdims=True)
    o = _dot(attn, vv_ref[0], 1, 0)
    h = _ln(o + xc, g1_ref[...], b1_ref[...])
    d = _dot(h, wd_ref[...], 1, 1) + bd_ref[...]
    x2 = _ln(d + h, g2_ref[...], b2_ref[...])
    x3 = x2 + xc
    G = BM // C
    wrow = jnp.concatenate([wfc_ref[...]] * G, axis=1)
    rows = lax.broadcasted_iota(jnp.int32, (G, BM), 0)
    lanes = lax.broadcasted_iota(jnp.int32, (G, BM), 1)
    wsel = jnp.where(lanes // C == rows,
                     jnp.broadcast_to(wrow, (G, BM)), 0.0)
    s_ref[0] = _dot(wsel, x3, 1, 0) + bfc_ref[0, 0]


def _attn(flat, Km, Vm, Wq, bq, Wd, bd, g1, b1, g2, b2, Wfc, bfc):
    return pl.pallas_call(
        _attn_body,
        grid=(B, L // BM),
        in_specs=[pl.BlockSpec((1, BM, N), lambda b, i: (b, i, 0)),
                  pl.BlockSpec((1, L, N), lambda b, i: (b, 0, 0)),
                  pl.BlockSpec((1, L, N), lambda b, i: (b, 0, 0)),
                  pl.BlockSpec((N, N), lambda b, i: (0, 0)),
                  pl.BlockSpec((1, N), lambda b, i: (0, 0)),
                  pl.BlockSpec((N, N), lambda b, i: (0, 0)),
                  pl.BlockSpec((1, N), lambda b, i: (0, 0)),
                  pl.BlockSpec((1, N), lambda b, i: (0, 0)),
                  pl.BlockSpec((1, N), lambda b, i: (0, 0)),
                  pl.BlockSpec((1, N), lambda b, i: (0, 0)),
                  pl.BlockSpec((1, N), lambda b, i: (0, 0)),
                  pl.BlockSpec((1, C), lambda b, i: (0, 0)),
                  pl.BlockSpec((1, 1), lambda b, i: (0, 0))],
        out_specs=pl.BlockSpec((1, BM // C, N), lambda b, i: (b, i, 0)),
        out_shape=jax.ShapeDtypeStruct((B, KT, N), jnp.float32),
    )(flat, Km, Vm, Wq, bq, Wd, bd, g1, b1, g2, b2, Wfc, bfc)


def _argmin_body(s_ref, gidx_ref):
    b = pl.program_id(0)
    i = pl.program_id(1)
    s = s_ref[0]
    lanes = lax.broadcasted_iota(jnp.int32, (BN, KT), 1)
    cols = lax.broadcasted_iota(jnp.int32, (BN, K), 1)
    rows = lax.broadcasted_iota(jnp.int32, (BN, 1), 0)
    base = (b * N + i * BN + rows) * KT

    def step(t, carry):
        sv, res = carry
        m = jnp.min(sv, axis=1, keepdims=True)
        cand = jnp.where(sv == m, lanes, KT)
        a = jnp.min(cand, axis=1, keepdims=True)
        res = jnp.where(cols == t, base + a, res)
        sv = jnp.where(lanes == a, jnp.inf, sv)
        return sv, res

    _, res = lax.fori_loop(0, K, step,
                           (s, jnp.zeros((BN, K), jnp.int32)))
    gidx_ref[0] = res


def _argmin(st):
    return pl.pallas_call(
        _argmin_body,
        grid=(B, N // BN),
        in_specs=[pl.BlockSpec((1, BN, KT), lambda b, i: (b, i, 0))],
        out_specs=pl.BlockSpec((1, BN, K), lambda b, i: (b, i, 0)),
        out_shape=jax.ShapeDtypeStruct((B, N, K), jnp.int32),
    )(st)


def kernel(x, Wq, bq, Wk, bk, Wv, bv, Wd, bd, Wfc, bfc, g1, beta1, g2, beta2):
    xt = jnp.transpose(x, (0, 2, 1))
    table = xt.reshape(B * N, C)
    idxg = _knn(x)
    feat = _sc_gather(table, idxg.reshape(-1, 128))
    xwn = _center(feat.reshape(B, N, KT, C), xt)
    flat = jnp.transpose(xwn, (0, 2, 3, 1)).reshape(B, L, N)
    Km, Vm = _kv(flat, Wk, bk.reshape(1, N), Wv, bv.reshape(1, N))
    s = _attn(flat, Km, Vm, Wq, bq.reshape(1, N), Wd, bd.reshape(1, N),
              g1.reshape(1, N), beta1.reshape(1, N),
              g2.reshape(1, N), beta2.reshape(1, N),
              Wfc, bfc.reshape(1, 1))
    st = jnp.transpose(s, (0, 2, 1))
    gidx = _argmin(st)
    ans_rows = _sc_gather(xwn.reshape(B * N * KT, C), gidx.reshape(-1, 128))
    ans = ans_rows.reshape(B, N, K, C)
    return jnp.transpose(ans, (0, 3, 2, 1))


# speedup vs baseline: 3.7469x; 3.7469x over previous
import functools

import jax
import jax.numpy as jnp
from jax import lax
from jax.experimental import pallas as pl
from jax.experimental.pallas import tpu as pltpu
from jax.experimental.pallas import tpu_sc as plsc

B, C, N = 4, 32, 1024
KT, K = 50, 20
L = KT * C
BM = 320
BN = 256


def _dot(a, b, ca, cb):
    return lax.dot_general(a, b, dimension_numbers=(((ca,), (cb,)), ((), ())),
                           preferred_element_type=jnp.float32)


def _ln(x, g, b):
    mu = jnp.mean(x, axis=-1, keepdims=True)
    var = jnp.mean((x - mu) ** 2, axis=-1, keepdims=True)
    return (x - mu) / jnp.sqrt(var + 1e-5) * g + b


def _knn_body(xf_ref, xb_ref, idx_ref):
    b = pl.program_id(0)
    xf = xf_ref[0]
    xb = xb_ref[0]
    xsq = jnp.sum(xf * xf, axis=0, keepdims=True)
    xx = _dot(xb, xf, 0, 0)
    sqb = jnp.sum(xb * xb, axis=0)[:, None]
    dist = 2.0 * xx - sqb - xsq
    lanes = lax.broadcasted_iota(jnp.int32, (BN, N), 1)
    cols = lax.broadcasted_iota(jnp.int32, (BN, KT), 1)
    boff = b * N

    def step(t, carry):
        d, res = carry
        m = jnp.max(d, axis=1, keepdims=True)
        cand = jnp.where(d == m, lanes, N)
        a = jnp.min(cand, axis=1, keepdims=True)
        res = jnp.where(cols == t, a + boff, res)
        d = jnp.where(lanes == a, -jnp.inf, d)
        return d, res

    _, res = lax.fori_loop(0, KT, step,
                           (dist, jnp.zeros((BN, KT), jnp.int32)))
    idx_ref[0] = res


def _knn(x):
    return pl.pallas_call(
        _knn_body,
        grid=(B, N // BN),
        in_specs=[pl.BlockSpec((1, C, N), lambda b, i: (b, 0, 0)),
                  pl.BlockSpec((1, C, BN), lambda b, i: (b, 0, i))],
        out_specs=pl.BlockSpec((1, BN, KT), lambda b, i: (b, i, 0)),
        out_shape=jax.ShapeDtypeStruct((B, N, KT), jnp.int32),
    )(x, x)


def _sc_gather(table, idx2d):
    info = plsc.get_sparse_core_info()
    nw = info.num_cores * info.num_subcores
    R = idx2d.shape[0]
    rpw = R // nw
    idx3d = idx2d.reshape(nw, rpw, 128)
    D = table.shape[1]

    @functools.partial(
        pl.kernel,
        mesh=plsc.VectorSubcoreMesh(core_axis_name="c", subcore_axis_name="s"),
        out_type=jax.ShapeDtypeStruct((R * 128, D), jnp.float32),
        scratch_types=[pltpu.VMEM((rpw, 128), jnp.int32),
                       pltpu.VMEM((128, D), jnp.float32),
                       pltpu.SemaphoreType.DMA],
    )
    def k(table_hbm, idx_hbm, out_hbm, idx_v, rows_v, sem):
        wid = lax.axis_index("s") * info.num_cores + lax.axis_index("c")
        pltpu.sync_copy(idx_hbm.at[wid], idx_v)

        def body(ci, _):
            pltpu.async_copy(table_hbm.at[idx_v.at[ci]], rows_v, sem).wait()
            pltpu.sync_copy(rows_v,
                            out_hbm.at[pl.ds((wid * rpw + ci) * 128, 128)])
            return 0

        lax.fori_loop(0, rpw, body, 0)

    return k(table, idx3d)


def _center_body(feat_ref, xt_ref, out_ref):
    xwn = feat_ref[0][:, :, :C] - xt_ref[0][:, None, :]
    out_ref[0] = jnp.concatenate(
        [xwn, jnp.zeros((BN, KT, 128 - C), jnp.float32)], axis=-1)


def _center(feat4, xt3):
    return pl.pallas_call(
        _center_body,
        grid=(B, N // BN),
        in_specs=[pl.BlockSpec((1, BN, KT, 128), lambda b, i: (b, i, 0, 0)),
                  pl.BlockSpec((1, BN, C), lambda b, i: (b, i, 0))],
        out_specs=pl.BlockSpec((1, BN, KT, 128), lambda b, i: (b, i, 0, 0)),
        out_shape=jax.ShapeDtypeStruct((B, N, KT, 128), jnp.float32),
    )(feat4, xt3)


def _kv_body(xc_ref, wk_ref, bk_ref, wv_ref, bv_ref, k_ref, v_ref):
    xc = xc_ref[0]
    k_ref[0] = _dot(xc, wk_ref[...], 1, 1) + bk_ref[...]
    v_ref[0] = _dot(xc, wv_ref[...], 1, 1) + bv_ref[...]


def _kv(flat, Wk, bk, Wv, bv):
    return pl.pallas_call(
        _kv_body,
        grid=(B, L // BM),
        in_specs=[pl.BlockSpec((1, BM, N), lambda b, i: (b, i, 0)),
                  pl.BlockSpec((N, N), lambda b, i: (0, 0)),
                  pl.BlockSpec((1, N), lambda b, i: (0, 0)),
                  pl.BlockSpec((N, N), lambda b, i: (0, 0)),
                  pl.BlockSpec((1, N), lambda b, i: (0, 0))],
        out_specs=[pl.BlockSpec((1, BM, N), lambda b, i: (b, i, 0)),
                   pl.BlockSpec((1, BM, N), lambda b, i: (b, i, 0))],
        out_shape=[jax.ShapeDtypeStruct((B, L, N), jnp.float32),
                   jax.ShapeDtypeStruct((B, L, N), jnp.float32)],
    )(flat, Wk, bk, Wv, bv)


def _attn_body(xc_ref, kk_ref, vv_ref, wq_ref, bq_ref, wd_ref, bd_ref,
               g1_ref, b1_ref, g2_ref, b2_ref, wfc_ref, bfc_ref, s_ref):
    xc = xc_ref[0]
    q = _dot(xc, wq_ref[...], 1, 1) + bq_ref[...]
    logits = _dot(q, kk_ref[0], 1, 1) * (1.0 / 32.0)
    m = jnp.max(logits, axis=1, keepdims=True)
    e = jnp.exp(logits - m)
    attn = e / jnp.sum(e, axis=1, keepdims=True)
    o = _dot(attn, vv_ref[0], 1, 0)
    h = _ln(o + xc, g1_ref[...], b1_ref[...])
    d = _dot(h, wd_ref[...], 1, 1) + bd_ref[...]
    x2 = _ln(d + h, g2_ref[...], b2_ref[...])
    x3 = x2 + xc
    G = BM // C
    wrow = jnp.concatenate([wfc_ref[...]] * G, axis=1)
    rows = lax.broadcasted_iota(jnp.int32, (G, BM), 0)
    lanes = lax.broadcasted_iota(jnp.int32, (G, BM), 1)
    wsel = jnp.where(lanes // C == rows,
                     jnp.broadcast_to(wrow, (G, BM)), 0.0)
    s_ref[0, 0] = _dot(wsel, x3, 1, 0) + bfc_ref[0, 0]


def _attn(flat, Km, Vm, Wq, bq, Wd, bd, g1, b1, g2, b2, Wfc, bfc):
    return pl.pallas_call(
        _attn_body,
        grid=(B, L // BM),
        in_specs=[pl.BlockSpec((1, BM, N), lambda b, i: (b, i, 0)),
                  pl.BlockSpec((1, L, N), lambda b, i: (b, 0, 0)),
                  pl.BlockSpec((1, L, N), lambda b, i: (b, 0, 0)),
                  pl.BlockSpec((N, N), lambda b, i: (0, 0)),
                  pl.BlockSpec((1, N), lambda b, i: (0, 0)),
                  pl.BlockSpec((N, N), lambda b, i: (0, 0)),
                  pl.BlockSpec((1, N), lambda b, i: (0, 0)),
                  pl.BlockSpec((1, N), lambda b, i: (0, 0)),
                  pl.BlockSpec((1, N), lambda b, i: (0, 0)),
                  pl.BlockSpec((1, N), lambda b, i: (0, 0)),
                  pl.BlockSpec((1, N), lambda b, i: (0, 0)),
                  pl.BlockSpec((1, C), lambda b, i: (0, 0)),
                  pl.BlockSpec((1, 1), lambda b, i: (0, 0))],
        out_specs=pl.BlockSpec((1, 1, BM // C, N), lambda b, i: (b, i, 0, 0)),
        out_shape=jax.ShapeDtypeStruct((B, L // BM, BM // C, N), jnp.float32),
    )(flat, Km, Vm, Wq, bq, Wd, bd, g1, b1, g2, b2, Wfc, bfc)


def _argmin_body(s_ref, gidx_ref):
    b = pl.program_id(0)
    i = pl.program_id(1)
    s = s_ref[0]
    lanes = lax.broadcasted_iota(jnp.int32, (BN, KT), 1)
    cols = lax.broadcasted_iota(jnp.int32, (BN, K), 1)
    rows = lax.broadcasted_iota(jnp.int32, (BN, 1), 0)
    base = (b * N + i * BN + rows) * KT

    def step(t, carry):
        sv, res = carry
        m = jnp.min(sv, axis=1, keepdims=True)
        cand = jnp.where(sv == m, lanes, KT)
        a = jnp.min(cand, axis=1, keepdims=True)
        res = jnp.where(cols == t, base + a, res)
        sv = jnp.where(lanes == a, jnp.inf, sv)
        return sv, res

    _, res = lax.fori_loop(0, K, step,
                           (s, jnp.zeros((BN, K), jnp.int32)))
    gidx_ref[0] = res


def _argmin(st):
    return pl.pallas_call(
        _argmin_body,
        grid=(B, N // BN),
        in_specs=[pl.BlockSpec((1, BN, KT), lambda b, i: (b, i, 0))],
        out_specs=pl.BlockSpec((1, BN, K), lambda b, i: (b, i, 0)),
        out_shape=jax.ShapeDtypeStruct((B, N, K), jnp.int32),
    )(st)


def kernel(x, Wq, bq, Wk, bk, Wv, bv, Wd, bd, Wfc, bfc, g1, beta1, g2, beta2):
    xt = jnp.transpose(x, (0, 2, 1))
    table = jnp.pad(xt.reshape(B * N, C), ((0, 0), (0, 128 - C)))
    idxg = _knn(x)
    feat = _sc_gather(table, idxg.reshape(-1, 128))
    xwn = _center(feat.reshape(B, N, KT, 128), xt)
    flat = jnp.transpose(xwn[..., :C], (0, 2, 3, 1)).reshape(B, L, N)
    Km, Vm = _kv(flat, Wk, bk.reshape(1, N), Wv, bv.reshape(1, N))
    s = _attn(flat, Km, Vm, Wq, bq.reshape(1, N), Wd, bd.reshape(1, N),
              g1.reshape(1, N), beta1.reshape(1, N),
              g2.reshape(1, N), beta2.reshape(1, N),
              Wfc, bfc.reshape(1, 1))
    st = jnp.transpose(s.reshape(B, KT, N), (0, 2, 1))
    gidx = _argmin(st)
    ans_rows = _sc_gather(xwn.reshape(B * N * KT, 128), gidx.reshape(-1, 128))
    ans = ans_rows.reshape(B, N, K, 128)[..., :C]
    return jnp.transpose(ans, (0, 3, 2, 1))
